# baseline (speedup 1.0000x reference)
import functools

import jax
import jax.numpy as jnp
from jax import lax
from jax.experimental import pallas as pl
from jax.experimental.pallas import tpu as pltpu
from jax.experimental.pallas import tpu_sc as plsc

SEQ_SIZE = 200
EMBED_DIM = 64
BATCH = 4096


def _make_sc_gather(V, D, B, NW, b_per_w):
    mesh = plsc.VectorSubcoreMesh(core_axis_name="c", subcore_axis_name="s")
    chunk = D * b_per_w

    @functools.partial(
        pl.kernel,
        mesh=mesh,
        out_type=jax.ShapeDtypeStruct((NW, chunk), jnp.float32),
        scratch_types=[
            pltpu.VMEM((chunk,), jnp.int32),
            pltpu.VMEM((chunk,), jnp.float32),
            pltpu.SemaphoreType.DMA,
        ],
        compiler_params=pltpu.CompilerParams(use_tc_tiling_on_sc=False),
    )
    def gather_kernel(table_hbm, idx_hbm, out_hbm, idx_v, vals_v, sem):
        wid = lax.axis_index("s") * 2 + lax.axis_index("c")
        pltpu.sync_copy(idx_hbm.at[wid], idx_v)
        pltpu.async_copy(table_hbm.at[idx_v], vals_v, sem).wait()
        pltpu.sync_copy(vals_v, out_hbm.at[wid])

    return gather_kernel


def _bcast_add_body(g_ref, posb_ref, out_ref):
    g = g_ref[...]
    pb = posb_ref[...]
    out_ref[...] = pb + g[None, :, :]


def kernel(inputs, token_table, pos_table):
    V, D = token_table.shape
    B = inputs.shape[0]
    info = plsc.get_sparse_core_info()
    NW = info.num_cores * info.num_subcores
    b_per_w = B // NW

    idx = inputs.reshape(B).astype(jnp.int32)

    table_flat = token_table.T.reshape(V * D)

    idx2 = (jnp.arange(D, dtype=jnp.int32)[None, :, None] * V
            + idx.reshape(NW, 1, b_per_w)).reshape(NW, D * b_per_w)
    g_flat = _make_sc_gather(V, D, B, NW, b_per_w)(table_flat, idx2)
    gT3 = g_flat.reshape(NW, D, b_per_w)
    gT = jnp.transpose(gT3, (1, 0, 2)).reshape(D, B)

    BBL = 256
    posB = jnp.broadcast_to(pos_table[:, :, None], (SEQ_SIZE, D, BBL))
    P = pl.pallas_call(
        _bcast_add_body,
        grid=(B // BBL,),
        in_specs=[
            pl.BlockSpec((D, BBL), lambda i: (0, i)),
            pl.BlockSpec((SEQ_SIZE, D, BBL), lambda i: (0, 0, 0)),
        ],
        out_specs=pl.BlockSpec((SEQ_SIZE, D, BBL), lambda i: (0, 0, i)),
        out_shape=jax.ShapeDtypeStruct((SEQ_SIZE, D, B), jnp.float32),
    )(gT, posB)
    return jnp.transpose(P, (2, 0, 1))


# speedup vs baseline: 7.3579x; 7.3579x over previous
import functools

import jax
import jax.numpy as jnp
from jax import lax
from jax.experimental import pallas as pl
from jax.experimental.pallas import tpu as pltpu
from jax.experimental.pallas import tpu_sc as plsc

SEQ_SIZE = 200
EMBED_DIM = 64
BATCH = 4096


def _make_sc_gather(V, D, B, NW, b_per_w):
    mesh = plsc.VectorSubcoreMesh(core_axis_name="c", subcore_axis_name="s")

    @functools.partial(
        pl.kernel,
        mesh=mesh,
        out_type=jax.ShapeDtypeStruct((B, D), jnp.float32),
        scratch_types=[
            pltpu.VMEM((b_per_w,), jnp.int32),
            pltpu.VMEM((b_per_w, D), jnp.float32),
            pltpu.SemaphoreType.DMA,
        ],
        compiler_params=pltpu.CompilerParams(use_tc_tiling_on_sc=False),
    )
    def gather_kernel(table_hbm, idx_hbm, out_hbm, idx_v, rows_v, sem):
        wid = lax.axis_index("s") * 2 + lax.axis_index("c")
        base = wid * b_per_w
        pltpu.sync_copy(idx_hbm.at[pl.ds(base, b_per_w)], idx_v)
        pltpu.async_copy(table_hbm.at[idx_v], rows_v, sem).wait()
        pltpu.sync_copy(rows_v, out_hbm.at[pl.ds(base, b_per_w)])

    return gather_kernel


def _bcast_add_body(g_ref, posb_ref, out_ref):
    g = g_ref[...]
    pb = posb_ref[...]
    out_ref[...] = pb + g[None, :, :]


def kernel(inputs, token_table, pos_table):
    V, D = token_table.shape
    B = inputs.shape[0]
    info = plsc.get_sparse_core_info()
    NW = info.num_cores * info.num_subcores
    b_per_w = B // NW

    idx = inputs.reshape(B).astype(jnp.int32)
    gathered = _make_sc_gather(V, D, B, NW, b_per_w)(token_table, idx)
    gT = gathered.T

    BBL = 256
    posB = jnp.broadcast_to(pos_table[:, :, None], (SEQ_SIZE, D, BBL))
    P = pl.pallas_call(
        _bcast_add_body,
        grid=(B // BBL,),
        in_specs=[
            pl.BlockSpec((D, BBL), lambda i: (0, i)),
            pl.BlockSpec((SEQ_SIZE, D, BBL), lambda i: (0, 0, 0)),
        ],
        out_specs=pl.BlockSpec((SEQ_SIZE, D, BBL), lambda i: (0, 0, i)),
        out_shape=jax.ShapeDtypeStruct((SEQ_SIZE, D, B), jnp.float32),
    )(gT, posB)
    return jnp.transpose(P, (2, 0, 1))
